# baseline (speedup 1.0000x reference)
import functools

import jax
import jax.numpy as jnp
from jax import lax
from jax.experimental import pallas as pl
from jax.experimental.pallas import tpu as pltpu
from jax.experimental.pallas import tpu_sc as plsc

N_IN = 8000
N_OUT = 2000
N_TOT = N_IN + N_OUT
E = 320000
DOM = 3
CH = 128

NC = 2
NS = 16
L = 16
NW = NC * NS

B_CH = 64
EPT = 10240
E_PAD = NW * EPT
NCHUNK = EPT // B_CH
NPAIR = NCHUNK // 2
N_PAD = 10240
ROWS_PT = N_PAD // NS
VPR = CH // L

_ROW_BLK = 1024
_GRID = N_PAD // _ROW_BLK


def _pre_body(x_ref, pos_ref, wl_ref, bl_ref, w1_ref, b1_ref,
              a_ref, b_ref, v_ref):
    v = jnp.dot(x_ref[...], wl_ref[...],
                preferred_element_type=jnp.float32,
                precision=jax.lax.Precision.HIGHEST) + bl_ref[...]
    pos = pos_ref[...]
    a_ref[...] = jnp.dot(pos, w1_ref[0:DOM, :],
                         preferred_element_type=jnp.float32,
                precision=jax.lax.Precision.HIGHEST) + b1_ref[...]
    b_ref[...] = (jnp.dot(pos, w1_ref[DOM:2 * DOM, :],
                          preferred_element_type=jnp.float32,
                precision=jax.lax.Precision.HIGHEST)
                  + jnp.dot(v, w1_ref[2 * DOM:, :],
                            preferred_element_type=jnp.float32,
                precision=jax.lax.Precision.HIGHEST))
    v_ref[...] = v


_pre = pl.pallas_call(
    _pre_body,
    grid=(_GRID,),
    in_specs=[
        pl.BlockSpec((_ROW_BLK, 1), lambda i: (i, 0)),
        pl.BlockSpec((_ROW_BLK, DOM), lambda i: (i, 0)),
        pl.BlockSpec((1, CH), lambda i: (0, 0)),
        pl.BlockSpec((1, CH), lambda i: (0, 0)),
        pl.BlockSpec((2 * DOM + CH, CH), lambda i: (0, 0)),
        pl.BlockSpec((1, CH), lambda i: (0, 0)),
    ],
    out_specs=[
        pl.BlockSpec((_ROW_BLK, CH), lambda i: (i, 0)),
        pl.BlockSpec((_ROW_BLK, CH), lambda i: (i, 0)),
        pl.BlockSpec((_ROW_BLK, CH), lambda i: (i, 0)),
    ],
    out_shape=[
        jax.ShapeDtypeStruct((N_PAD, CH), jnp.float32),
        jax.ShapeDtypeStruct((N_PAD, CH), jnp.float32),
        jax.ShapeDtypeStruct((N_PAD, CH), jnp.float32),
    ],
)


def _post_body(s0_ref, s1_ref, h_ref, v_ref, w2_ref, b2_ref, wloc_ref,
               bias_ref, wproj_ref, bproj_ref, out_ref):
    s = s0_ref[...] + s1_ref[...]
    cnt = jnp.sum(h_ref[...], axis=0)[:, None]
    summed = jnp.dot(s, w2_ref[...],
                     preferred_element_type=jnp.float32,
                precision=jax.lax.Precision.HIGHEST) + cnt * b2_ref[...]
    aggr = summed / jnp.maximum(cnt, 1.0)
    w = aggr + jnp.dot(v_ref[...], wloc_ref[...],
                       preferred_element_type=jnp.float32,
                precision=jax.lax.Precision.HIGHEST) + bias_ref[...]
    out_ref[...] = jnp.maximum(
        jnp.dot(w, wproj_ref[...], preferred_element_type=jnp.float32,
                precision=jax.lax.Precision.HIGHEST)
        + bproj_ref[...], 0.0)


_post = pl.pallas_call(
    _post_body,
    grid=(_GRID,),
    in_specs=[
        pl.BlockSpec((_ROW_BLK, CH), lambda i: (i, 0)),
        pl.BlockSpec((_ROW_BLK, CH), lambda i: (i, 0)),
        pl.BlockSpec((NW, _ROW_BLK), lambda i: (0, i)),
        pl.BlockSpec((_ROW_BLK, CH), lambda i: (i, 0)),
        pl.BlockSpec((CH, CH), lambda i: (0, 0)),
        pl.BlockSpec((1, CH), lambda i: (0, 0)),
        pl.BlockSpec((CH, CH), lambda i: (0, 0)),
        pl.BlockSpec((1, CH), lambda i: (0, 0)),
        pl.BlockSpec((CH, CH), lambda i: (0, 0)),
        pl.BlockSpec((1, CH), lambda i: (0, 0)),
    ],
    out_specs=pl.BlockSpec((_ROW_BLK, CH), lambda i: (i, 0)),
    out_shape=jax.ShapeDtypeStruct((N_PAD, CH), jnp.float32),
)


def _edge_body(a_hbm, b_hbm, dst_hbm, src_hbm, zero_hbm, out_hbm, cnt_hbm,
               dst0, src0, dst1, src1, a0, b0, a1, b1, hist_v, s_sh,
               sa0, sb0, sa1, sb1):
    c = lax.axis_index("c")
    s = lax.axis_index("s")
    wid = c * NS + s
    zero16 = jnp.zeros((L,), jnp.float32)

    def hzero(i, carry):
        hist_v[pl.ds(i * L, L)] = zero16
        return carry

    lax.fori_loop(0, N_PAD // L, hzero, 0)

    pltpu.sync_copy(zero_hbm, s_sh.at[pl.ds(s * ROWS_PT, ROWS_PT)])

    plsc.subcore_barrier()

    base = wid * EPT

    def hist(dstb):
        def hupd(q, hcarry):
            vdst = dstb[pl.ds(q * L, L)]
            run, last = plsc.scan_count(vdst)
            plsc.addupdate_scatter(hist_v, [vdst], run.astype(jnp.float32),
                                   mask=last)
            return hcarry

        lax.fori_loop(0, B_CH // L, hupd, 0)

    def consume(dstb, ab, bb):
        def comp(r, icarry):
            for j in range(VPR):
                va = ab[r, pl.ds(j * L, L)]
                vb = bb[r, pl.ds(j * L, L)]
                ab[r, pl.ds(j * L, L)] = jnp.maximum(va + vb,
                                                     jnp.float32(0.0))
            return icarry

        lax.fori_loop(0, B_CH, comp, 0)
        pltpu.sync_copy(ab, s_sh.at[dstb], add=True)

    def pair(p, carry):
        off0 = base + (2 * p) * B_CH
        off1 = off0 + B_CH
        pltpu.sync_copy(dst_hbm.at[pl.ds(off0, B_CH)], dst0)
        pltpu.sync_copy(src_hbm.at[pl.ds(off0, B_CH)], src0)
        ga0 = pltpu.async_copy(a_hbm.at[dst0], a0, sa0)
        gb0 = pltpu.async_copy(b_hbm.at[src0], b0, sb0)
        pltpu.sync_copy(dst_hbm.at[pl.ds(off1, B_CH)], dst1)
        pltpu.sync_copy(src_hbm.at[pl.ds(off1, B_CH)], src1)
        ga1 = pltpu.async_copy(a_hbm.at[dst1], a1, sa1)
        gb1 = pltpu.async_copy(b_hbm.at[src1], b1, sb1)
        hist(dst0)
        hist(dst1)
        ga0.wait()
        gb0.wait()
        consume(dst0, a0, b0)
        ga1.wait()
        gb1.wait()
        consume(dst1, a1, b1)
        return carry

    lax.fori_loop(0, NPAIR, pair, 0)

    plsc.subcore_barrier()

    rows = pl.ds(s * ROWS_PT, ROWS_PT)
    pltpu.sync_copy(s_sh.at[rows], out_hbm.at[c, rows])
    pltpu.sync_copy(hist_v, cnt_hbm.at[wid])


_edge = functools.partial(
    pl.kernel,
    out_type=(
        jax.ShapeDtypeStruct((NC, N_PAD, CH), jnp.float32),
        jax.ShapeDtypeStruct((NW, N_PAD), jnp.float32),
    ),
    mesh=plsc.VectorSubcoreMesh(core_axis_name="c", subcore_axis_name="s"),
    compiler_params=pltpu.CompilerParams(needs_layout_passes=False),
    scratch_types=[
        pltpu.VMEM((B_CH,), jnp.int32),
        pltpu.VMEM((B_CH,), jnp.int32),
        pltpu.VMEM((B_CH,), jnp.int32),
        pltpu.VMEM((B_CH,), jnp.int32),
        pltpu.VMEM((B_CH, CH), jnp.float32),
        pltpu.VMEM((B_CH, CH), jnp.float32),
        pltpu.VMEM((B_CH, CH), jnp.float32),
        pltpu.VMEM((B_CH, CH), jnp.float32),
        pltpu.VMEM((N_PAD,), jnp.float32),
        pltpu.VMEM_SHARED((N_PAD, CH), jnp.float32),
        pltpu.SemaphoreType.DMA,
        pltpu.SemaphoreType.DMA,
        pltpu.SemaphoreType.DMA,
        pltpu.SemaphoreType.DMA,
    ],
)(_edge_body)


def kernel(x, pos_x, pos_y, edge_index, W_lift, b_lift, W1, b1, W2, b2,
           W_loc, bias, W_proj, b_proj):
    pos = jnp.concatenate(
        [pos_x, pos_y, jnp.zeros((N_PAD - N_TOT, DOM), dtype=pos_x.dtype)],
        axis=0)
    x_full = jnp.concatenate(
        [x, jnp.zeros((N_PAD - N_IN, x.shape[1]), dtype=x.dtype)], axis=0)
    pad_idx = jnp.full((E_PAD - E,), N_PAD - 1, dtype=jnp.int32)
    src = jnp.concatenate([edge_index[0], pad_idx])
    dst = jnp.concatenate([edge_index[1], pad_idx])

    a_tab, b_tab, v = _pre(x_full, pos, W_lift, b_lift.reshape(1, CH), W1,
                           b1.reshape(1, CH))
    zeros_blk = jnp.zeros((ROWS_PT, CH), jnp.float32)
    part, hist = _edge(a_tab, b_tab, dst, src, zeros_blk)
    w = _post(part[0], part[1], hist, v, W2,
              b2.reshape(1, CH), W_loc, bias.reshape(1, CH), W_proj,
              b_proj.reshape(1, CH))
    return w[:N_IN], w[N_IN:N_TOT]


# speedup vs baseline: 2.6772x; 2.6772x over previous
import functools

import jax
import jax.numpy as jnp
from jax import lax
from jax.experimental import pallas as pl
from jax.experimental.pallas import tpu as pltpu
from jax.experimental.pallas import tpu_sc as plsc

N_IN = 8000
N_OUT = 2000
N_TOT = N_IN + N_OUT
E = 320000
DOM = 3
CH = 128

NC = 2
NS = 16
L = 16
NW = NC * NS

B_CH = 64
EPT = 10240
E_PAD = NW * EPT
NCHUNK = EPT // B_CH
NPAIR = NCHUNK // 2
N_PAD = 10240
ROWS_PT = N_PAD // NS
VPR = CH // L

_ROW_BLK = 1024
_GRID = N_PAD // _ROW_BLK


def _pre_body(x_ref, pos_ref, wl_ref, bl_ref, w1_ref, b1_ref,
              a_ref, b_ref, v_ref):
    v = jnp.dot(x_ref[...], wl_ref[...],
                preferred_element_type=jnp.float32,
                precision=jax.lax.Precision.HIGHEST) + bl_ref[...]
    pos = pos_ref[...]
    a_ref[...] = jnp.dot(pos, w1_ref[0:DOM, :],
                         preferred_element_type=jnp.float32,
                precision=jax.lax.Precision.HIGHEST) + b1_ref[...]
    b_ref[...] = (jnp.dot(pos, w1_ref[DOM:2 * DOM, :],
                          preferred_element_type=jnp.float32,
                precision=jax.lax.Precision.HIGHEST)
                  + jnp.dot(v, w1_ref[2 * DOM:, :],
                            preferred_element_type=jnp.float32,
                precision=jax.lax.Precision.HIGHEST))
    v_ref[...] = v


_pre = pl.pallas_call(
    _pre_body,
    grid=(_GRID,),
    in_specs=[
        pl.BlockSpec((_ROW_BLK, 1), lambda i: (i, 0)),
        pl.BlockSpec((_ROW_BLK, DOM), lambda i: (i, 0)),
        pl.BlockSpec((1, CH), lambda i: (0, 0)),
        pl.BlockSpec((1, CH), lambda i: (0, 0)),
        pl.BlockSpec((2 * DOM + CH, CH), lambda i: (0, 0)),
        pl.BlockSpec((1, CH), lambda i: (0, 0)),
    ],
    out_specs=[
        pl.BlockSpec((_ROW_BLK, CH), lambda i: (i, 0)),
        pl.BlockSpec((_ROW_BLK, CH), lambda i: (i, 0)),
        pl.BlockSpec((_ROW_BLK, CH), lambda i: (i, 0)),
    ],
    out_shape=[
        jax.ShapeDtypeStruct((N_PAD, CH), jnp.float32),
        jax.ShapeDtypeStruct((N_PAD, CH), jnp.float32),
        jax.ShapeDtypeStruct((N_PAD, CH), jnp.float32),
    ],
)


def _post_body(s0_ref, s1_ref, h_ref, v_ref, w2_ref, b2_ref, wloc_ref,
               bias_ref, wproj_ref, bproj_ref, out_ref):
    s = s0_ref[...] + s1_ref[...]
    cnt = jnp.sum(h_ref[...], axis=0)[:, None]
    summed = jnp.dot(s, w2_ref[...],
                     preferred_element_type=jnp.float32,
                precision=jax.lax.Precision.HIGHEST) + cnt * b2_ref[...]
    aggr = summed / jnp.maximum(cnt, 1.0)
    w = aggr + jnp.dot(v_ref[...], wloc_ref[...],
                       preferred_element_type=jnp.float32,
                precision=jax.lax.Precision.HIGHEST) + bias_ref[...]
    out_ref[...] = jnp.maximum(
        jnp.dot(w, wproj_ref[...], preferred_element_type=jnp.float32,
                precision=jax.lax.Precision.HIGHEST)
        + bproj_ref[...], 0.0)


_post = pl.pallas_call(
    _post_body,
    grid=(_GRID,),
    in_specs=[
        pl.BlockSpec((_ROW_BLK, CH), lambda i: (i, 0)),
        pl.BlockSpec((_ROW_BLK, CH), lambda i: (i, 0)),
        pl.BlockSpec((NW, _ROW_BLK), lambda i: (0, i)),
        pl.BlockSpec((_ROW_BLK, CH), lambda i: (i, 0)),
        pl.BlockSpec((CH, CH), lambda i: (0, 0)),
        pl.BlockSpec((1, CH), lambda i: (0, 0)),
        pl.BlockSpec((CH, CH), lambda i: (0, 0)),
        pl.BlockSpec((1, CH), lambda i: (0, 0)),
        pl.BlockSpec((CH, CH), lambda i: (0, 0)),
        pl.BlockSpec((1, CH), lambda i: (0, 0)),
    ],
    out_specs=pl.BlockSpec((_ROW_BLK, CH), lambda i: (i, 0)),
    out_shape=jax.ShapeDtypeStruct((N_PAD, CH), jnp.float32),
)


def _edge_body(a_hbm, b_hbm, dst_hbm, src_hbm, zero_hbm, out_hbm, cnt_hbm,
               dst0, src0, dst1, src1, a0, b0, a1, b1, hist_v, s_sh,
               sa0, sb0, sa1, sb1):
    c = lax.axis_index("c")
    s = lax.axis_index("s")
    wid = c * NS + s
    zero16 = jnp.zeros((L,), jnp.float32)

    def hzero(i, carry):
        hist_v[pl.ds(i * L, L)] = zero16
        return carry

    lax.fori_loop(0, N_PAD // L, hzero, 0)

    pltpu.sync_copy(zero_hbm, s_sh.at[pl.ds(s * ROWS_PT, ROWS_PT)])

    plsc.subcore_barrier()

    base = wid * EPT

    def hist(dstb):
        def hupd(q, hcarry):
            vdst = dstb[pl.ds(q * L, L)]
            run, last = plsc.scan_count(vdst)
            plsc.addupdate_scatter(hist_v, [vdst], run.astype(jnp.float32),
                                   mask=last)
            return hcarry

        lax.fori_loop(0, B_CH // L, hupd, 0)

    def consume(dstb, ab, bb):
        def comp(r, icarry):
            for j in range(VPR):
                va = ab[r, pl.ds(j * L, L)]
                vb = bb[r, pl.ds(j * L, L)]
                ab[r, pl.ds(j * L, L)] = jnp.maximum(va + vb,
                                                     jnp.float32(0.0))
            return icarry

        lax.fori_loop(0, B_CH, comp, 0)
        pltpu.sync_copy(ab, s_sh.at[dstb], add=True)

    def pair(p, carry):
        off0 = base + (2 * p) * B_CH
        off1 = off0 + B_CH
        pltpu.sync_copy(dst_hbm.at[pl.ds(off0, B_CH)], dst0)
        pltpu.sync_copy(src_hbm.at[pl.ds(off0, B_CH)], src0)
        ga0 = pltpu.async_copy(a_hbm.at[dst0], a0, sa0)
        gb0 = pltpu.async_copy(b_hbm.at[src0], b0, sb0)
        pltpu.sync_copy(dst_hbm.at[pl.ds(off1, B_CH)], dst1)
        pltpu.sync_copy(src_hbm.at[pl.ds(off1, B_CH)], src1)
        ga1 = pltpu.async_copy(a_hbm.at[dst1], a1, sa1)
        gb1 = pltpu.async_copy(b_hbm.at[src1], b1, sb1)
        hist(dst0)
        hist(dst1)
        ga0.wait()
        gb0.wait()
        consume(dst0, a0, b0)
        ga1.wait()
        gb1.wait()
        consume(dst1, a1, b1)
        return carry

    lax.fori_loop(0, NPAIR, pair, 0)

    plsc.subcore_barrier()

    rows = pl.ds(s * ROWS_PT, ROWS_PT)
    pltpu.sync_copy(s_sh.at[rows], out_hbm.at[c, rows])
    pltpu.sync_copy(hist_v, cnt_hbm.at[wid])


_edge = functools.partial(
    pl.kernel,
    out_type=(
        jax.ShapeDtypeStruct((NC, N_PAD, CH), jnp.float32),
        jax.ShapeDtypeStruct((NW, N_PAD), jnp.float32),
    ),
    mesh=plsc.VectorSubcoreMesh(core_axis_name="c", subcore_axis_name="s"),
    compiler_params=pltpu.CompilerParams(needs_layout_passes=False),
    scratch_types=[
        pltpu.VMEM((B_CH,), jnp.int32),
        pltpu.VMEM((B_CH,), jnp.int32),
        pltpu.VMEM((B_CH,), jnp.int32),
        pltpu.VMEM((B_CH,), jnp.int32),
        pltpu.VMEM((B_CH, CH), jnp.float32),
        pltpu.VMEM((B_CH, CH), jnp.float32),
        pltpu.VMEM((B_CH, CH), jnp.float32),
        pltpu.VMEM((B_CH, CH), jnp.float32),
        pltpu.VMEM((N_PAD,), jnp.float32),
        pltpu.VMEM_SHARED((N_PAD, CH), jnp.float32),
        pltpu.SemaphoreType.DMA,
        pltpu.SemaphoreType.DMA,
        pltpu.SemaphoreType.DMA,
        pltpu.SemaphoreType.DMA,
    ],
)(_edge_body)


def kernel(x, pos_x, pos_y, edge_index, W_lift, b_lift, W1, b1, W2, b2,
           W_loc, bias, W_proj, b_proj):
    pos = jnp.concatenate(
        [pos_x, pos_y, jnp.zeros((N_PAD - N_TOT, DOM), dtype=pos_x.dtype)],
        axis=0)
    x_full = jnp.concatenate(
        [x, jnp.zeros((N_PAD - N_IN, x.shape[1]), dtype=x.dtype)], axis=0)
    pad_idx = N_TOT + jnp.arange(E_PAD - E, dtype=jnp.int32) % (N_PAD - N_TOT)
    src = jnp.concatenate([edge_index[0], pad_idx])
    dst = jnp.concatenate([edge_index[1], pad_idx])

    a_tab, b_tab, v = _pre(x_full, pos, W_lift, b_lift.reshape(1, CH), W1,
                           b1.reshape(1, CH))
    zeros_blk = jnp.zeros((ROWS_PT, CH), jnp.float32)
    part, hist = _edge(a_tab, b_tab, dst, src, zeros_blk)
    w = _post(part[0], part[1], hist, v, W2,
              b2.reshape(1, CH), W_loc, bias.reshape(1, CH), W_proj,
              b_proj.reshape(1, CH))
    return w[:N_IN], w[N_IN:N_TOT]


# speedup vs baseline: 2.8387x; 1.0603x over previous
import functools

import jax
import jax.numpy as jnp
from jax import lax
from jax.experimental import pallas as pl
from jax.experimental.pallas import tpu as pltpu
from jax.experimental.pallas import tpu_sc as plsc

N_IN = 8000
N_OUT = 2000
N_TOT = N_IN + N_OUT
E = 320000
DOM = 3
CH = 128

NC = 2
NS = 16
L = 16
NW = NC * NS

B_CH = 64
EPT = 10240
E_PAD = NW * EPT
NCHUNK = EPT // B_CH
NPAIR = NCHUNK // 2
N_PAD = 10240
ROWS_PT = N_PAD // NS
VPR = CH // L

_ROW_BLK = 1024
_GRID = N_PAD // _ROW_BLK


def _pre_body(x_ref, pos_ref, wl_ref, bl_ref, w1_ref, b1_ref,
              a_ref, b_ref, v_ref):
    v = jnp.dot(x_ref[...], wl_ref[...],
                preferred_element_type=jnp.float32,
                precision=jax.lax.Precision.HIGHEST) + bl_ref[...]
    pos = pos_ref[...]
    a_ref[...] = jnp.dot(pos, w1_ref[0:DOM, :],
                         preferred_element_type=jnp.float32,
                precision=jax.lax.Precision.HIGHEST) + b1_ref[...]
    b_ref[...] = (jnp.dot(pos, w1_ref[DOM:2 * DOM, :],
                          preferred_element_type=jnp.float32,
                precision=jax.lax.Precision.HIGHEST)
                  + jnp.dot(v, w1_ref[2 * DOM:, :],
                            preferred_element_type=jnp.float32,
                precision=jax.lax.Precision.HIGHEST))
    v_ref[...] = v


_pre = pl.pallas_call(
    _pre_body,
    grid=(_GRID,),
    in_specs=[
        pl.BlockSpec((_ROW_BLK, 1), lambda i: (i, 0)),
        pl.BlockSpec((_ROW_BLK, DOM), lambda i: (i, 0)),
        pl.BlockSpec((1, CH), lambda i: (0, 0)),
        pl.BlockSpec((1, CH), lambda i: (0, 0)),
        pl.BlockSpec((2 * DOM + CH, CH), lambda i: (0, 0)),
        pl.BlockSpec((1, CH), lambda i: (0, 0)),
    ],
    out_specs=[
        pl.BlockSpec((_ROW_BLK, CH), lambda i: (i, 0)),
        pl.BlockSpec((_ROW_BLK, CH), lambda i: (i, 0)),
        pl.BlockSpec((_ROW_BLK, CH), lambda i: (i, 0)),
    ],
    out_shape=[
        jax.ShapeDtypeStruct((N_PAD, CH), jnp.float32),
        jax.ShapeDtypeStruct((N_PAD, CH), jnp.float32),
        jax.ShapeDtypeStruct((N_PAD, CH), jnp.float32),
    ],
)


def _post_body(s0_ref, s1_ref, h_ref, v_ref, w2_ref, b2_ref, wloc_ref,
               bias_ref, wproj_ref, bproj_ref, out_ref):
    s = s0_ref[...] + s1_ref[...]
    cnt = jnp.sum(h_ref[...], axis=0)[:, None]
    summed = jnp.dot(s, w2_ref[...],
                     preferred_element_type=jnp.float32,
                precision=jax.lax.Precision.HIGHEST) + cnt * b2_ref[...]
    aggr = summed / jnp.maximum(cnt, 1.0)
    w = aggr + jnp.dot(v_ref[...], wloc_ref[...],
                       preferred_element_type=jnp.float32,
                precision=jax.lax.Precision.HIGHEST) + bias_ref[...]
    out_ref[...] = jnp.maximum(
        jnp.dot(w, wproj_ref[...], preferred_element_type=jnp.float32,
                precision=jax.lax.Precision.HIGHEST)
        + bproj_ref[...], 0.0)


_post = pl.pallas_call(
    _post_body,
    grid=(_GRID,),
    in_specs=[
        pl.BlockSpec((_ROW_BLK, CH), lambda i: (i, 0)),
        pl.BlockSpec((_ROW_BLK, CH), lambda i: (i, 0)),
        pl.BlockSpec((NW, _ROW_BLK), lambda i: (0, i)),
        pl.BlockSpec((_ROW_BLK, CH), lambda i: (i, 0)),
        pl.BlockSpec((CH, CH), lambda i: (0, 0)),
        pl.BlockSpec((1, CH), lambda i: (0, 0)),
        pl.BlockSpec((CH, CH), lambda i: (0, 0)),
        pl.BlockSpec((1, CH), lambda i: (0, 0)),
        pl.BlockSpec((CH, CH), lambda i: (0, 0)),
        pl.BlockSpec((1, CH), lambda i: (0, 0)),
    ],
    out_specs=pl.BlockSpec((_ROW_BLK, CH), lambda i: (i, 0)),
    out_shape=jax.ShapeDtypeStruct((N_PAD, CH), jnp.float32),
)


def _edge_body(a_hbm, b_hbm, dst_hbm, src_hbm, zero_hbm, out_hbm, cnt_hbm,
               dst0, src0, dst1, src1, a0, b0, a1, b1, hist_v, s_sh,
               sa0, sb0, sa1, sb1):
    c = lax.axis_index("c")
    s = lax.axis_index("s")
    wid = c * NS + s
    zero16 = jnp.zeros((L,), jnp.float32)

    def hzero(i, carry):
        hist_v[pl.ds(i * L, L)] = zero16
        return carry

    lax.fori_loop(0, N_PAD // L, hzero, 0)

    pltpu.sync_copy(zero_hbm, s_sh.at[pl.ds(s * ROWS_PT, ROWS_PT)])

    plsc.subcore_barrier()

    base = wid * EPT

    def hist(dstb):
        def hupd(q, hcarry):
            vdst = dstb[pl.ds(q * L, L)]
            run, last = plsc.scan_count(vdst)
            plsc.addupdate_scatter(hist_v, [vdst], run.astype(jnp.float32),
                                   mask=last)
            return hcarry

        lax.fori_loop(0, B_CH // L, hupd, 0)

    def consume(dstb, ab, bb):
        def comp(r, icarry):
            for j in range(VPR):
                va = ab[r, pl.ds(j * L, L)]
                vb = bb[r, pl.ds(j * L, L)]
                ab[r, pl.ds(j * L, L)] = jnp.maximum(va + vb,
                                                     jnp.float32(0.0))
            return icarry

        lax.fori_loop(0, B_CH, comp, 0)
        pltpu.sync_copy(ab, s_sh.at[dstb], add=True)

    def issue(k, dstb, srcb, ab, bb, sa, sb):
        off = base + k * B_CH
        pltpu.sync_copy(dst_hbm.at[pl.ds(off, B_CH)], dstb)
        pltpu.sync_copy(src_hbm.at[pl.ds(off, B_CH)], srcb)
        pltpu.async_copy(a_hbm.at[dstb], ab, sa)
        pltpu.async_copy(b_hbm.at[srcb], bb, sb)

    issue(0, dst0, src0, a0, b0, sa0, sb0)
    issue(1, dst1, src1, a1, b1, sa1, sb1)

    def pair(p, carry):
        hist(dst0)
        pltpu.make_async_copy(a_hbm.at[dst0], a0, sa0).wait()
        pltpu.make_async_copy(b_hbm.at[src0], b0, sb0).wait()
        consume(dst0, a0, b0)

        @pl.when(p < NPAIR - 1)
        def _():
            issue(2 * p + 2, dst0, src0, a0, b0, sa0, sb0)

        hist(dst1)
        pltpu.make_async_copy(a_hbm.at[dst1], a1, sa1).wait()
        pltpu.make_async_copy(b_hbm.at[src1], b1, sb1).wait()
        consume(dst1, a1, b1)

        @pl.when(p < NPAIR - 1)
        def _():
            issue(2 * p + 3, dst1, src1, a1, b1, sa1, sb1)

        return carry

    lax.fori_loop(0, NPAIR, pair, 0)

    plsc.subcore_barrier()

    rows = pl.ds(s * ROWS_PT, ROWS_PT)
    pltpu.sync_copy(s_sh.at[rows], out_hbm.at[c, rows])
    pltpu.sync_copy(hist_v, cnt_hbm.at[wid])


_edge = functools.partial(
    pl.kernel,
    out_type=(
        jax.ShapeDtypeStruct((NC, N_PAD, CH), jnp.float32),
        jax.ShapeDtypeStruct((NW, N_PAD), jnp.float32),
    ),
    mesh=plsc.VectorSubcoreMesh(core_axis_name="c", subcore_axis_name="s"),
    compiler_params=pltpu.CompilerParams(needs_layout_passes=False),
    scratch_types=[
        pltpu.VMEM((B_CH,), jnp.int32),
        pltpu.VMEM((B_CH,), jnp.int32),
        pltpu.VMEM((B_CH,), jnp.int32),
        pltpu.VMEM((B_CH,), jnp.int32),
        pltpu.VMEM((B_CH, CH), jnp.float32),
        pltpu.VMEM((B_CH, CH), jnp.float32),
        pltpu.VMEM((B_CH, CH), jnp.float32),
        pltpu.VMEM((B_CH, CH), jnp.float32),
        pltpu.VMEM((N_PAD,), jnp.float32),
        pltpu.VMEM_SHARED((N_PAD, CH), jnp.float32),
        pltpu.SemaphoreType.DMA,
        pltpu.SemaphoreType.DMA,
        pltpu.SemaphoreType.DMA,
        pltpu.SemaphoreType.DMA,
    ],
)(_edge_body)


def kernel(x, pos_x, pos_y, edge_index, W_lift, b_lift, W1, b1, W2, b2,
           W_loc, bias, W_proj, b_proj):
    pos = jnp.concatenate(
        [pos_x, pos_y, jnp.zeros((N_PAD - N_TOT, DOM), dtype=pos_x.dtype)],
        axis=0)
    x_full = jnp.concatenate(
        [x, jnp.zeros((N_PAD - N_IN, x.shape[1]), dtype=x.dtype)], axis=0)
    pad_idx = N_TOT + jnp.arange(E_PAD - E, dtype=jnp.int32) % (N_PAD - N_TOT)
    src = jnp.concatenate([edge_index[0], pad_idx])
    dst = jnp.concatenate([edge_index[1], pad_idx])

    a_tab, b_tab, v = _pre(x_full, pos, W_lift, b_lift.reshape(1, CH), W1,
                           b1.reshape(1, CH))
    zeros_blk = jnp.zeros((ROWS_PT, CH), jnp.float32)
    part, hist = _edge(a_tab, b_tab, dst, src, zeros_blk)
    w = _post(part[0], part[1], hist, v, W2,
              b2.reshape(1, CH), W_loc, bias.reshape(1, CH), W_proj,
              b_proj.reshape(1, CH))
    return w[:N_IN], w[N_IN:N_TOT]


# speedup vs baseline: 3.2098x; 1.1307x over previous
import functools

import jax
import jax.numpy as jnp
from jax import lax
from jax.experimental import pallas as pl
from jax.experimental.pallas import tpu as pltpu
from jax.experimental.pallas import tpu_sc as plsc

N_IN = 8000
N_OUT = 2000
N_TOT = N_IN + N_OUT
E = 320000
DOM = 3
CH = 128

NC = 2
NS = 16
L = 16
NW = NC * NS

B_CH = 64
EPT = 10240
E_PAD = NW * EPT
NCHUNK = EPT // B_CH
NPAIR = NCHUNK // 2
N_PAD = 10240
ROWS_PT = N_PAD // NS
VPR = CH // L

_ROW_BLK = 1024
_GRID = N_PAD // _ROW_BLK


def _pre_body(x_ref, pos_ref, wl_ref, bl_ref, w1_ref, b1_ref,
              a_ref, b_ref, v_ref):
    v = jnp.dot(x_ref[...], wl_ref[...],
                preferred_element_type=jnp.float32,
                precision=jax.lax.Precision.HIGHEST) + bl_ref[...]
    pos = pos_ref[...]
    a_ref[...] = jnp.dot(pos, w1_ref[0:DOM, :],
                         preferred_element_type=jnp.float32,
                precision=jax.lax.Precision.HIGHEST) + b1_ref[...]
    b_ref[...] = (jnp.dot(pos, w1_ref[DOM:2 * DOM, :],
                          preferred_element_type=jnp.float32,
                precision=jax.lax.Precision.HIGHEST)
                  + jnp.dot(v, w1_ref[2 * DOM:, :],
                            preferred_element_type=jnp.float32,
                precision=jax.lax.Precision.HIGHEST))
    v_ref[...] = v


_pre = pl.pallas_call(
    _pre_body,
    grid=(_GRID,),
    in_specs=[
        pl.BlockSpec((_ROW_BLK, 1), lambda i: (i, 0)),
        pl.BlockSpec((_ROW_BLK, DOM), lambda i: (i, 0)),
        pl.BlockSpec((1, CH), lambda i: (0, 0)),
        pl.BlockSpec((1, CH), lambda i: (0, 0)),
        pl.BlockSpec((2 * DOM + CH, CH), lambda i: (0, 0)),
        pl.BlockSpec((1, CH), lambda i: (0, 0)),
    ],
    out_specs=[
        pl.BlockSpec((_ROW_BLK, CH), lambda i: (i, 0)),
        pl.BlockSpec((_ROW_BLK, CH), lambda i: (i, 0)),
        pl.BlockSpec((_ROW_BLK, CH), lambda i: (i, 0)),
    ],
    out_shape=[
        jax.ShapeDtypeStruct((N_PAD, CH), jnp.float32),
        jax.ShapeDtypeStruct((N_PAD, CH), jnp.float32),
        jax.ShapeDtypeStruct((N_PAD, CH), jnp.float32),
    ],
)


def _post_body(s0_ref, s1_ref, h_ref, v_ref, w2_ref, b2_ref, wloc_ref,
               bias_ref, wproj_ref, bproj_ref, out_ref):
    s = s0_ref[...] + s1_ref[...]
    cnt = jnp.sum(h_ref[...], axis=0)[:, None]
    summed = jnp.dot(s, w2_ref[...],
                     preferred_element_type=jnp.float32,
                precision=jax.lax.Precision.HIGHEST) + cnt * b2_ref[...]
    aggr = summed / jnp.maximum(cnt, 1.0)
    w = aggr + jnp.dot(v_ref[...], wloc_ref[...],
                       preferred_element_type=jnp.float32,
                precision=jax.lax.Precision.HIGHEST) + bias_ref[...]
    out_ref[...] = jnp.maximum(
        jnp.dot(w, wproj_ref[...], preferred_element_type=jnp.float32,
                precision=jax.lax.Precision.HIGHEST)
        + bproj_ref[...], 0.0)


_post = pl.pallas_call(
    _post_body,
    grid=(_GRID,),
    in_specs=[
        pl.BlockSpec((_ROW_BLK, CH), lambda i: (i, 0)),
        pl.BlockSpec((_ROW_BLK, CH), lambda i: (i, 0)),
        pl.BlockSpec((NW, _ROW_BLK), lambda i: (0, i)),
        pl.BlockSpec((_ROW_BLK, CH), lambda i: (i, 0)),
        pl.BlockSpec((CH, CH), lambda i: (0, 0)),
        pl.BlockSpec((1, CH), lambda i: (0, 0)),
        pl.BlockSpec((CH, CH), lambda i: (0, 0)),
        pl.BlockSpec((1, CH), lambda i: (0, 0)),
        pl.BlockSpec((CH, CH), lambda i: (0, 0)),
        pl.BlockSpec((1, CH), lambda i: (0, 0)),
    ],
    out_specs=pl.BlockSpec((_ROW_BLK, CH), lambda i: (i, 0)),
    out_shape=jax.ShapeDtypeStruct((N_PAD, CH), jnp.float32),
)


def _edge_body(a_hbm, b_hbm, e_hbm, zero_hbm, out_hbm, cnt_hbm,
               ei0, ei1, a0, b0, a1, b1, hist_v, s_sh,
               sa0, sb0, sa1, sb1):
    c = lax.axis_index("c")
    s = lax.axis_index("s")
    wid = c * NS + s
    zero16 = jnp.zeros((L,), jnp.float32)

    def hzero(i, carry):
        hist_v[pl.ds(i * L, L)] = zero16
        return carry

    lax.fori_loop(0, N_PAD // L, hzero, 0)

    pltpu.sync_copy(zero_hbm, s_sh.at[pl.ds(s * ROWS_PT, ROWS_PT)])

    plsc.subcore_barrier()

    base = wid * EPT

    def hist(dstb):
        def hupd(q, hcarry):
            vdst = dstb[pl.ds(q * L, L)]
            run, last = plsc.scan_count(vdst)
            plsc.addupdate_scatter(hist_v, [vdst], run.astype(jnp.float32),
                                   mask=last)
            return hcarry

        lax.fori_loop(0, B_CH // L, hupd, 0)

    def consume(dstb, ab, bb):
        def comp(r, icarry):
            for j in range(VPR):
                va = ab[r, pl.ds(j * L, L)]
                vb = bb[r, pl.ds(j * L, L)]
                ab[r, pl.ds(j * L, L)] = jnp.maximum(va + vb,
                                                     jnp.float32(0.0))
            return icarry

        lax.fori_loop(0, B_CH, comp, 0)
        pltpu.sync_copy(ab, s_sh.at[dstb], add=True)

    def issue(k, eib, ab, bb, sa, sb):
        pltpu.sync_copy(e_hbm.at[wid * NCHUNK + k], eib)
        pltpu.async_copy(a_hbm.at[eib.at[0]], ab, sa)
        pltpu.async_copy(b_hbm.at[eib.at[1]], bb, sb)

    issue(0, ei0, a0, b0, sa0, sb0)
    issue(1, ei1, a1, b1, sa1, sb1)

    def pair(p, carry):
        hist(ei0.at[0])
        pltpu.make_async_copy(a_hbm.at[ei0.at[0]], a0, sa0).wait()
        pltpu.make_async_copy(b_hbm.at[ei0.at[1]], b0, sb0).wait()
        consume(ei0.at[0], a0, b0)

        @pl.when(p < NPAIR - 1)
        def _():
            issue(2 * p + 2, ei0, a0, b0, sa0, sb0)

        hist(ei1.at[0])
        pltpu.make_async_copy(a_hbm.at[ei1.at[0]], a1, sa1).wait()
        pltpu.make_async_copy(b_hbm.at[ei1.at[1]], b1, sb1).wait()
        consume(ei1.at[0], a1, b1)

        @pl.when(p < NPAIR - 1)
        def _():
            issue(2 * p + 3, ei1, a1, b1, sa1, sb1)

        return carry

    lax.fori_loop(0, NPAIR, pair, 0)

    plsc.subcore_barrier()

    rows = pl.ds(s * ROWS_PT, ROWS_PT)
    pltpu.sync_copy(s_sh.at[rows], out_hbm.at[c, rows])
    pltpu.sync_copy(hist_v, cnt_hbm.at[wid])


_edge = functools.partial(
    pl.kernel,
    out_type=(
        jax.ShapeDtypeStruct((NC, N_PAD, CH), jnp.float32),
        jax.ShapeDtypeStruct((NW, N_PAD), jnp.float32),
    ),
    mesh=plsc.VectorSubcoreMesh(core_axis_name="c", subcore_axis_name="s"),
    compiler_params=pltpu.CompilerParams(needs_layout_passes=False),
    scratch_types=[
        pltpu.VMEM((2, B_CH), jnp.int32),
        pltpu.VMEM((2, B_CH), jnp.int32),
        pltpu.VMEM((B_CH, CH), jnp.float32),
        pltpu.VMEM((B_CH, CH), jnp.float32),
        pltpu.VMEM((B_CH, CH), jnp.float32),
        pltpu.VMEM((B_CH, CH), jnp.float32),
        pltpu.VMEM((N_PAD,), jnp.float32),
        pltpu.VMEM_SHARED((N_PAD, CH), jnp.float32),
        pltpu.SemaphoreType.DMA,
        pltpu.SemaphoreType.DMA,
        pltpu.SemaphoreType.DMA,
        pltpu.SemaphoreType.DMA,
    ],
)(_edge_body)


def kernel(x, pos_x, pos_y, edge_index, W_lift, b_lift, W1, b1, W2, b2,
           W_loc, bias, W_proj, b_proj):
    pos = jnp.concatenate(
        [pos_x, pos_y, jnp.zeros((N_PAD - N_TOT, DOM), dtype=pos_x.dtype)],
        axis=0)
    x_full = jnp.concatenate(
        [x, jnp.zeros((N_PAD - N_IN, x.shape[1]), dtype=x.dtype)], axis=0)
    pad_idx = N_TOT + jnp.arange(E_PAD - E, dtype=jnp.int32) % (N_PAD - N_TOT)
    src = jnp.concatenate([edge_index[0], pad_idx])
    dst = jnp.concatenate([edge_index[1], pad_idx])
    eidx = jnp.stack([dst.reshape(NW * NCHUNK, B_CH),
                      src.reshape(NW * NCHUNK, B_CH)], axis=1)

    a_tab, b_tab, v = _pre(x_full, pos, W_lift, b_lift.reshape(1, CH), W1,
                           b1.reshape(1, CH))
    zeros_blk = jnp.zeros((ROWS_PT, CH), jnp.float32)
    part, hist = _edge(a_tab, b_tab, eidx, zeros_blk)
    w = _post(part[0], part[1], hist, v, W2,
              b2.reshape(1, CH), W_loc, bias.reshape(1, CH), W_proj,
              b_proj.reshape(1, CH))
    return w[:N_IN], w[N_IN:N_TOT]
